# baseline (speedup 1.0000x reference)
import functools
import math

import jax
import jax.numpy as jnp
from jax import lax
from jax.experimental import pallas as pl
from jax.experimental.pallas import tpu as pltpu
from jax.experimental.pallas import tpu_sc as plsc

VOCAB = 1000000
D_MODEL = 64
BATCH = 4096
HIST = 200

NUM_CORES = 2
NUM_SUBCORES = 16
NW = NUM_CORES * NUM_SUBCORES

B = BATCH * HIST
BPW = B // NW
CHUNK = 640
NCHUNK = BPW // CHUNK
LANES = 16
VPR = D_MODEL // LANES

SCALE = math.sqrt(D_MODEL)

_mesh = plsc.VectorSubcoreMesh(core_axis_name="c", subcore_axis_name="s")


@functools.partial(
    pl.kernel,
    out_type=jax.ShapeDtypeStruct((B, D_MODEL), jnp.float32),
    mesh=_mesh,
    scratch_types=[
        pltpu.VMEM((BPW,), jnp.int32),
        pltpu.VMEM((CHUNK, D_MODEL), jnp.float32),
        pltpu.SemaphoreType.DMA,
    ],
    compiler_params=pltpu.CompilerParams(use_tc_tiling_on_sc=False),
)
def _emb_lookup(x_hbm, table_hbm, out_hbm, idx_v, buf, sem):
    wid = lax.axis_index("s") * NUM_CORES + lax.axis_index("c")
    base = wid * BPW
    pltpu.sync_copy(x_hbm.at[pl.ds(base, BPW)], idx_v)

    def step(ci, carry):
        row0 = ci * CHUNK
        pltpu.async_copy(
            table_hbm.at[idx_v.at[pl.ds(row0, CHUNK)]], buf, sem
        ).wait()

        def scale_row(r, c2):
            for j in range(VPR):
                buf[r, pl.ds(j * LANES, LANES)] = (
                    buf[r, pl.ds(j * LANES, LANES)] * SCALE
                )
            return c2

        lax.fori_loop(0, CHUNK, scale_row, 0)

        pltpu.sync_copy(buf, out_hbm.at[pl.ds(base + row0, CHUNK)])
        return carry

    lax.fori_loop(0, NCHUNK, step, 0)


def kernel(x, table):
    out = _emb_lookup(x.reshape(-1), table)
    return out.reshape(BATCH, HIST, D_MODEL)


# speedup vs baseline: 1.1104x; 1.1104x over previous
import functools
import math

import jax
import jax.numpy as jnp
from jax import lax
from jax.experimental import pallas as pl
from jax.experimental.pallas import tpu as pltpu
from jax.experimental.pallas import tpu_sc as plsc

VOCAB = 1000000
D_MODEL = 64
BATCH = 4096
HIST = 200

NUM_CORES = 2
NUM_SUBCORES = 16
NW = NUM_CORES * NUM_SUBCORES

B = BATCH * HIST
BPW = B // NW
CHUNK = 320
NCHUNK = BPW // CHUNK
LANES = 16
VPR = D_MODEL // LANES

SCALE = math.sqrt(D_MODEL)

_mesh = plsc.VectorSubcoreMesh(core_axis_name="c", subcore_axis_name="s")


@functools.partial(
    pl.kernel,
    out_type=jax.ShapeDtypeStruct((B, D_MODEL), jnp.float32),
    mesh=_mesh,
    scratch_types=[
        pltpu.VMEM((BPW,), jnp.int32),
        pltpu.VMEM((CHUNK, D_MODEL), jnp.float32),
        pltpu.VMEM((CHUNK, D_MODEL), jnp.float32),
        pltpu.VMEM((CHUNK, D_MODEL), jnp.float32),
        pltpu.VMEM((CHUNK, D_MODEL), jnp.float32),
        pltpu.SemaphoreType.DMA,
        pltpu.SemaphoreType.DMA,
        pltpu.SemaphoreType.DMA,
        pltpu.SemaphoreType.DMA,
    ],
    compiler_params=pltpu.CompilerParams(use_tc_tiling_on_sc=False),
)
def _emb_lookup(x_hbm, table_hbm, out_hbm,
                idx_v, g0, g1, o0, o1, gs0, gs1, os0, os1):
    wid = lax.axis_index("s") * NUM_CORES + lax.axis_index("c")
    base = wid * BPW
    gbufs, obufs = (g0, g1), (o0, o1)
    gsems, osems = (gs0, gs1), (os0, os1)

    pltpu.sync_copy(x_hbm.at[pl.ds(base, BPW)], idx_v)

    def gather_desc(ci, b):
        return pltpu.make_async_copy(
            table_hbm.at[idx_v.at[pl.ds(ci * CHUNK, CHUNK)]],
            gbufs[b], gsems[b])

    def out_desc(ci, b):
        return pltpu.make_async_copy(
            obufs[b], out_hbm.at[pl.ds(base + ci * CHUNK, CHUNK)], osems[b])

    gather_desc(0, 0).start()
    gather_desc(1, 1).start()

    @pl.loop(0, NCHUNK, step=2)
    def _visit(ci0):
        for b in range(2):
            ci = ci0 + b
            gather_desc(ci, b).wait()

            @pl.when(ci >= 2)
            def _():
                out_desc(ci, b).wait()

            @plsc.parallel_loop(0, CHUNK, unroll=8)
            def _(r):
                for j in range(VPR):
                    obufs[b][r, pl.ds(j * LANES, LANES)] = (
                        gbufs[b][r, pl.ds(j * LANES, LANES)] * SCALE)

            @pl.when(ci + 2 < NCHUNK)
            def _():
                gather_desc(ci + 2, b).start()

            out_desc(ci, b).start()

    for b in range(2):
        out_desc(NCHUNK - 2 + b, b).wait()


def kernel(x, table):
    out = _emb_lookup(x.reshape(-1), table)
    return out.reshape(BATCH, HIST, D_MODEL)


# speedup vs baseline: 1.1528x; 1.0381x over previous
import functools
import math

import jax
import jax.numpy as jnp
from jax import lax
from jax.experimental import pallas as pl
from jax.experimental.pallas import tpu as pltpu
from jax.experimental.pallas import tpu_sc as plsc

VOCAB = 1000000
D_MODEL = 64
BATCH = 4096
HIST = 200

NUM_CORES = 2
NUM_SUBCORES = 16
NW = NUM_CORES * NUM_SUBCORES

LANES = 16
WPAD = 128
BBLK = BATCH // NW
HC = HIST // 8
SCALE = math.sqrt(D_MODEL)

_mesh = plsc.VectorSubcoreMesh(core_axis_name="c", subcore_axis_name="s")


@functools.partial(
    pl.kernel,
    out_type=jax.ShapeDtypeStruct((HIST, 8, NW, 8, BBLK), jnp.float32),
    mesh=_mesh,
    scratch_types=[
        pltpu.VMEM((HC, 8, BBLK), jnp.int32),
        pltpu.VMEM((BBLK, WPAD), jnp.float32),
        pltpu.VMEM((BBLK, WPAD), jnp.float32),
        pltpu.VMEM((8, 8, BBLK), jnp.float32),
        pltpu.VMEM((8, 8, BBLK), jnp.float32),
        pltpu.SemaphoreType.DMA,
        pltpu.SemaphoreType.DMA,
        pltpu.SemaphoreType.DMA,
        pltpu.SemaphoreType.DMA,
    ],
    compiler_params=pltpu.CompilerParams(
        use_tc_tiling_on_sc=False, needs_layout_passes=False),
)
def _emb_lookup(xq_hbm, tp_hbm, out_hbm,
                idx_v, g0, g1, s0, s1, gs0, gs1, os0, os1):
    wid = lax.axis_index("s") * NUM_CORES + lax.axis_index("c")
    gbufs, sbufs = (g0, g1), (s0, s1)
    gsems, osems = (gs0, gs1), (os0, os1)

    pltpu.sync_copy(xq_hbm.at[:, wid], idx_v)

    def gather_desc(h, b):
        hc = lax.shift_right_logical(h, 3)
        hl = lax.bitwise_and(h, 7)
        return pltpu.make_async_copy(
            tp_hbm.at[idx_v.at[hc, hl]], gbufs[b], gsems[b])

    def out_desc(h, b):
        return pltpu.make_async_copy(
            sbufs[b], out_hbm.at[h, :, wid], osems[b])

    row_ids = [lax.iota(jnp.int32, 16) + jb * LANES for jb in range(8)]

    gather_desc(0, 0).start()
    gather_desc(1, 1).start()

    @pl.loop(0, HIST, step=2)
    def _visit(h0):
        for b in range(2):
            h = h0 + b
            gather_desc(h, b).wait()

            @pl.when(h >= 2)
            def _():
                out_desc(h, b).wait()

            @plsc.parallel_loop(0, D_MODEL, unroll=2)
            def _(d):
                col = jnp.full((LANES,), 0, jnp.int32) + d
                tr = lax.shift_right_logical(d, 3)
                dlo = lax.bitwise_and(d, 7)
                for jb in range(8):
                    v = plsc.load_gather(gbufs[b], [row_ids[jb], col])
                    sbufs[b][tr, dlo, pl.ds(jb * LANES, LANES)] = v * SCALE

            @pl.when(h + 2 < HIST)
            def _():
                gather_desc(h + 2, b).start()

            out_desc(h, b).start()

    for b in range(2):
        out_desc(HIST - 2 + b, b).wait()


def kernel(x, table):
    xq = x.T.reshape(HC, 8, NW, BBLK).transpose(0, 2, 1, 3)
    tp = jnp.pad(table, ((0, 0), (0, WPAD - D_MODEL)))
    lin = _emb_lookup(xq, tp)
    return lin.transpose(2, 4, 0, 1, 3).reshape(BATCH, HIST, D_MODEL)


# speedup vs baseline: 1.7665x; 1.5324x over previous
import functools
import math

import jax
import jax.numpy as jnp
from jax import lax
from jax.experimental import pallas as pl
from jax.experimental.pallas import tpu as pltpu
from jax.experimental.pallas import tpu_sc as plsc

VOCAB = 1000000
D_MODEL = 64
BATCH = 4096
HIST = 200

NUM_CORES = 2
NUM_SUBCORES = 16
NW = NUM_CORES * NUM_SUBCORES

LANES = 16
WPAD = 128
BBLK = BATCH // NW
HC = HIST // 8
SCALE = math.sqrt(D_MODEL)

_mesh = plsc.VectorSubcoreMesh(core_axis_name="c", subcore_axis_name="s")


@functools.partial(
    pl.kernel,
    out_type=jax.ShapeDtypeStruct((HIST, 8, NW, 8, BBLK), jnp.float32),
    mesh=_mesh,
    scratch_types=[
        pltpu.VMEM((HC, 8, BBLK), jnp.int32),
        pltpu.VMEM((BBLK, WPAD), jnp.float32),
        pltpu.VMEM((BBLK, WPAD), jnp.float32),
        pltpu.VMEM((8, 8, BBLK + 1), jnp.float32),
        pltpu.VMEM((8, 8, BBLK + 1), jnp.float32),
        pltpu.SemaphoreType.DMA,
        pltpu.SemaphoreType.DMA,
        pltpu.SemaphoreType.DMA,
        pltpu.SemaphoreType.DMA,
    ],
    compiler_params=pltpu.CompilerParams(
        use_tc_tiling_on_sc=False, needs_layout_passes=False),
)
def _emb_lookup(xq_hbm, tp_hbm, out_hbm,
                idx_v, g0, g1, s0, s1, gs0, gs1, os0, os1):
    wid = lax.axis_index("s") * NUM_CORES + lax.axis_index("c")
    gbufs, sbufs = (g0, g1), (s0, s1)
    gsems, osems = (gs0, gs1), (os0, os1)

    pltpu.sync_copy(xq_hbm.at[:, wid], idx_v)

    def gather_desc(h, b):
        hc = lax.shift_right_logical(h, 3)
        hl = lax.bitwise_and(h, 7)
        return pltpu.make_async_copy(
            tp_hbm.at[idx_v.at[hc, hl]], gbufs[b], gsems[b])

    def out_desc(h, b):
        return pltpu.make_async_copy(
            sbufs[b].at[:, :, pl.ds(0, BBLK)], out_hbm.at[h, :, wid], osems[b])

    iota = lax.iota(jnp.int32, LANES)
    tr_ids = [lax.shift_right_logical(iota + jd * LANES, 3) for jd in range(4)]
    dlo_ids = [lax.bitwise_and(iota + jd * LANES, 7) for jd in range(4)]
    zero_v = jnp.full((LANES,), 0, jnp.int32)

    gather_desc(0, 0).start()
    gather_desc(1, 1).start()

    @pl.loop(0, HIST, step=2)
    def _visit(h0):
        for b in range(2):
            h = h0 + b
            gather_desc(h, b).wait()

            @pl.when(h >= 2)
            def _():
                out_desc(h, b).wait()

            @plsc.parallel_loop(0, BBLK, unroll=2)
            def _(r):
                blo = zero_v + r
                for jd in range(4):
                    v = gbufs[b][r, pl.ds(jd * LANES, LANES)] * SCALE
                    plsc.store_scatter(
                        sbufs[b], [tr_ids[jd], dlo_ids[jd], blo], v)

            @pl.when(h + 2 < HIST)
            def _():
                gather_desc(h + 2, b).start()

            out_desc(h, b).start()

    for b in range(2):
        out_desc(HIST - 2 + b, b).wait()


def kernel(x, table):
    xq = x.T.reshape(HC, 8, NW, BBLK).transpose(0, 2, 1, 3)
    tp = jnp.pad(table, ((0, 0), (0, WPAD - D_MODEL)))
    lin = _emb_lookup(xq, tp)
    return lin.transpose(2, 4, 0, 1, 3).reshape(BATCH, HIST, D_MODEL)


# speedup vs baseline: 1.7730x; 1.0037x over previous
import functools
import math

import jax
import jax.numpy as jnp
from jax import lax
from jax.experimental import pallas as pl
from jax.experimental.pallas import tpu as pltpu
from jax.experimental.pallas import tpu_sc as plsc

VOCAB = 1000000
D_MODEL = 64
BATCH = 4096
HIST = 200

NUM_CORES = 2
NUM_SUBCORES = 16
NW = NUM_CORES * NUM_SUBCORES

LANES = 16
WPAD = 64
BBLK = BATCH // NW
HC = HIST // 8
SCALE = math.sqrt(D_MODEL)

_mesh = plsc.VectorSubcoreMesh(core_axis_name="c", subcore_axis_name="s")


@functools.partial(
    pl.kernel,
    out_type=jax.ShapeDtypeStruct((HIST, 8, NW, 8, BBLK), jnp.float32),
    mesh=_mesh,
    scratch_types=[
        pltpu.VMEM((HC, 8, BBLK), jnp.int32),
        pltpu.VMEM((BBLK, WPAD), jnp.float32),
        pltpu.VMEM((BBLK, WPAD), jnp.float32),
        pltpu.VMEM((8, 8, BBLK + 1), jnp.float32),
        pltpu.VMEM((8, 8, BBLK + 1), jnp.float32),
        pltpu.SemaphoreType.DMA,
        pltpu.SemaphoreType.DMA,
        pltpu.SemaphoreType.DMA,
        pltpu.SemaphoreType.DMA,
    ],
    compiler_params=pltpu.CompilerParams(
        use_tc_tiling_on_sc=False, needs_layout_passes=False),
)
def _emb_lookup(xq_hbm, tp_hbm, out_hbm,
                idx_v, g0, g1, s0, s1, gs0, gs1, os0, os1):
    wid = lax.axis_index("s") * NUM_CORES + lax.axis_index("c")
    gbufs, sbufs = (g0, g1), (s0, s1)
    gsems, osems = (gs0, gs1), (os0, os1)

    pltpu.sync_copy(xq_hbm.at[:, wid], idx_v)

    def gather_desc(h, b):
        hc = lax.shift_right_logical(h, 3)
        hl = lax.bitwise_and(h, 7)
        return pltpu.make_async_copy(
            tp_hbm.at[idx_v.at[hc, hl]], gbufs[b], gsems[b])

    def out_desc(h, b):
        return pltpu.make_async_copy(
            sbufs[b].at[:, :, pl.ds(0, BBLK)], out_hbm.at[h, :, wid], osems[b])

    iota = lax.iota(jnp.int32, LANES)
    tr_ids = [lax.shift_right_logical(iota + jd * LANES, 3) for jd in range(4)]
    dlo_ids = [lax.bitwise_and(iota + jd * LANES, 7) for jd in range(4)]
    zero_v = jnp.full((LANES,), 0, jnp.int32)

    gather_desc(0, 0).start()
    gather_desc(1, 1).start()

    @pl.loop(0, HIST, step=2)
    def _visit(h0):
        for b in range(2):
            h = h0 + b
            gather_desc(h, b).wait()

            @pl.when(h >= 2)
            def _():
                out_desc(h, b).wait()

            @plsc.parallel_loop(0, BBLK, unroll=2)
            def _(r):
                blo = zero_v + r
                for jd in range(4):
                    v = gbufs[b][r, pl.ds(jd * LANES, LANES)] * SCALE
                    plsc.store_scatter(
                        sbufs[b], [tr_ids[jd], dlo_ids[jd], blo], v)

            @pl.when(h + 2 < HIST)
            def _():
                gather_desc(h + 2, b).start()

            out_desc(h, b).start()

    for b in range(2):
        out_desc(HIST - 2 + b, b).wait()


def kernel(x, table):
    xq = x.T.reshape(HC, 8, NW, BBLK).transpose(0, 2, 1, 3)
    lin = _emb_lookup(xq, table)
    return lin.transpose(2, 4, 0, 1, 3).reshape(BATCH, HIST, D_MODEL)
